# Initial kernel scaffold; baseline (speedup 1.0000x reference)
import jax
import jax.numpy as jnp
from jax.experimental import pallas as pl


def kernel(actor_input, edge_index, W_g1, b_g1, W_g2, b_g2, W_g3, b_g3, W_f1, b_f1, W_f2, b_f2, W_f3, b_f3):
    raise NotImplementedError("write your pallas kernel here")


# speedup vs baseline: 11.0680x; 11.0680x over previous
import functools

import jax
import jax.numpy as jnp
from jax import lax
from jax.experimental import pallas as pl
from jax.experimental.pallas import tpu as pltpu
from jax.experimental.pallas import tpu_sc as plsc

N = 10000
E = 320000
D = 128
NW = 32
CHUNK = 128
CPW = 80
EPW = CHUNK * CPW
E_PAD = NW * EPW
NROWS = N + 112
ZROWS = NROWS // 16
NDUMMY = 112

_mesh = plsc.VectorSubcoreMesh(core_axis_name="c", subcore_axis_name="s")


@functools.partial(
    pl.kernel,
    out_type=(
        jax.ShapeDtypeStruct((2, NROWS, 16), jnp.float32),
        jax.ShapeDtypeStruct((2, NROWS, 16), jnp.float32),
    ),
    mesh=_mesh,
    scratch_types=[
        pltpu.VMEM((CPW, CHUNK), jnp.int32),
        pltpu.VMEM((CPW, CHUNK), jnp.int32),
        pltpu.VMEM((CHUNK, 16), jnp.float32),
        pltpu.VMEM_SHARED((NROWS, 16), jnp.float32),
        pltpu.VMEM_SHARED((NROWS, 16), jnp.float32),
    ],
)
def _deg_pass(srcb_hbm, dstb_hbm, degs_hbm, degd_hbm,
              src_v, dst_v, ones_v, degs_sh, degd_sh):
    c = lax.axis_index("c")
    s = lax.axis_index("s")
    w = c * 16 + s

    vzero = jnp.zeros((16,), jnp.float32)
    vone = jnp.ones((16,), jnp.float32)

    def fill(i, _):
        ones_v[i, pl.ds(0, 16)] = vzero
        return 0

    lax.fori_loop(0, CHUNK, fill, 0)

    zb = s * ZROWS
    for tgt in (degs_sh, degd_sh):
        for k in range(4):
            pltpu.sync_copy(ones_v, tgt.at[pl.ds(zb + k * 128, 128)])
        pltpu.sync_copy(ones_v.at[pl.ds(0, ZROWS - 512)],
                        tgt.at[pl.ds(zb + 512, ZROWS - 512)])

    def refill(i, _):
        ones_v[i, pl.ds(0, 16)] = vone
        return 0

    lax.fori_loop(0, CHUNK, refill, 0)

    pltpu.sync_copy(srcb_hbm.at[w], src_v)
    pltpu.sync_copy(dstb_hbm.at[w], dst_v)
    plsc.subcore_barrier()

    def chunk(j, _):
        pltpu.sync_copy(ones_v, degs_sh.at[src_v.at[j]], add=True)
        pltpu.sync_copy(ones_v, degd_sh.at[dst_v.at[j]], add=True)
        return 0

    lax.fori_loop(0, CPW, chunk, 0)
    plsc.subcore_barrier()

    pltpu.sync_copy(degs_sh.at[pl.ds(zb, ZROWS)], degs_hbm.at[c, pl.ds(zb, ZROWS)])
    pltpu.sync_copy(degd_sh.at[pl.ds(zb, ZROWS)], degd_hbm.at[c, pl.ds(zb, ZROWS)])


@functools.partial(
    pl.kernel,
    out_type=jax.ShapeDtypeStruct((2, NROWS, D), jnp.float32),
    mesh=_mesh,
    scratch_types=[
        pltpu.VMEM((CPW, CHUNK), jnp.int32),
        pltpu.VMEM((CPW, CHUNK), jnp.int32),
        pltpu.VMEM((CHUNK, D), jnp.float32),
        pltpu.VMEM_SHARED((NROWS, D), jnp.float32),
        pltpu.SemaphoreType.DMA,
    ],
)
def _edge_pass(hh_hbm, srcb_hbm, dstb_hbm, out_hbm,
               src_v, dst_v, rows, agg_sh, sem):
    c = lax.axis_index("c")
    s = lax.axis_index("s")
    w = c * 16 + s

    vzero = jnp.zeros((16,), jnp.float32)

    def zrow(i, _):
        for k in range(8):
            rows[i, pl.ds(k * 16, 16)] = vzero
        return 0

    lax.fori_loop(0, CHUNK, zrow, 0)

    zb = s * ZROWS
    for k in range(4):
        pltpu.sync_copy(rows, agg_sh.at[pl.ds(zb + k * 128, 128)])
    pltpu.sync_copy(rows.at[pl.ds(0, ZROWS - 512)],
                    agg_sh.at[pl.ds(zb + 512, ZROWS - 512)])

    pltpu.sync_copy(srcb_hbm.at[w], src_v)
    pltpu.sync_copy(dstb_hbm.at[w], dst_v)
    plsc.subcore_barrier()

    def chunk(j, _):
        pltpu.async_copy(hh_hbm.at[src_v.at[j]], rows, sem).wait()
        pltpu.sync_copy(rows, agg_sh.at[dst_v.at[j]], add=True)
        return 0

    lax.fori_loop(0, CPW, chunk, 0)
    plsc.subcore_barrier()

    pltpu.sync_copy(agg_sh.at[pl.ds(zb, ZROWS)], out_hbm.at[c, pl.ds(zb, ZROWS)])


_BLK = 1000
_GRID = N // _BLK


def _col0_norm(dref):
    d = dref[0, :, 0:1] + dref[1, :, 0:1] + 1.0
    return lax.rsqrt(d)


def _scale_body(x_ref, ds_ref, o_ref):
    o_ref[...] = x_ref[...] * _col0_norm(ds_ref)


def _layer_body(p_ref, hh_ref, dd_ref, ds_ref, w_ref, b_ref, o_ref):
    agg = (p_ref[0] + p_ref[1] + hh_ref[...]) * _col0_norm(dd_ref)
    h = jnp.dot(agg, w_ref[...], preferred_element_type=jnp.float32) + b_ref[...]
    h = jnp.maximum(h, 0.0)
    o_ref[...] = h * _col0_norm(ds_ref)


def _final_body(p_ref, hh_ref, dd_ref, wg_ref, bg_ref,
                w1_ref, b1_ref, w2_ref, b2_ref, w3_ref, b3_ref,
                prob_ref, g_ref):
    agg = (p_ref[0] + p_ref[1] + hh_ref[...]) * _col0_norm(dd_ref)
    z = jnp.dot(agg, wg_ref[...], preferred_element_type=jnp.float32) + bg_ref[...]
    g = jax.nn.sigmoid(z) + 1e-8
    g_ref[...] = g
    x = jnp.maximum(jnp.dot(g, w1_ref[...], preferred_element_type=jnp.float32)
                    + b1_ref[...], 0.0)
    x = jnp.maximum(jnp.dot(x, w2_ref[...], preferred_element_type=jnp.float32)
                    + b2_ref[...], 0.0)
    lg = jnp.dot(x, w3_ref[...], preferred_element_type=jnp.float32) + b3_ref[...]
    m = jnp.max(lg, axis=-1, keepdims=True)
    e = jnp.exp(lg - m)
    prob_ref[...] = e / jnp.sum(e, axis=-1, keepdims=True)


def _rows_spec(width):
    return pl.BlockSpec((_BLK, width), lambda i: (i, 0))


def _deg_spec():
    return pl.BlockSpec((2, _BLK, 16), lambda i: (0, i, 0))


def _parts_spec():
    return pl.BlockSpec((2, _BLK, D), lambda i: (0, i, 0))


def _w_spec(r, k):
    return pl.BlockSpec((r, k), lambda i: (0, 0))


def _tc_scale(x, degs):
    return pl.pallas_call(
        _scale_body,
        grid=(_GRID,),
        in_specs=[_rows_spec(D), _deg_spec()],
        out_specs=_rows_spec(D),
        out_shape=jax.ShapeDtypeStruct((N, D), jnp.float32),
    )(x, degs)


def _tc_layer(parts, hh, degd, degs, w, b):
    return pl.pallas_call(
        _layer_body,
        grid=(_GRID,),
        in_specs=[_parts_spec(), _rows_spec(D), _deg_spec(), _deg_spec(),
                  _w_spec(D, D), _w_spec(1, D)],
        out_specs=_rows_spec(D),
        out_shape=jax.ShapeDtypeStruct((N, D), jnp.float32),
    )(parts, hh, degd, degs, w, b)


def _tc_final(parts, hh, degd, wg, bg, w1, b1, w2, b2, w3, b3):
    return pl.pallas_call(
        _final_body,
        grid=(_GRID,),
        in_specs=[_parts_spec(), _rows_spec(D), _deg_spec(),
                  _w_spec(D, D), _w_spec(1, D),
                  _w_spec(D, 256), _w_spec(1, 256),
                  _w_spec(256, 256), _w_spec(1, 256),
                  _w_spec(256, 16), _w_spec(1, 16)],
        out_specs=[_rows_spec(16), _rows_spec(D)],
        out_shape=[jax.ShapeDtypeStruct((N, 16), jnp.float32),
                   jax.ShapeDtypeStruct((N, D), jnp.float32)],
    )(parts, hh, degd, wg, bg, w1, b1, w2, b2, w3, b3)


def kernel(actor_input, edge_index, W_g1, b_g1, W_g2, b_g2, W_g3, b_g3,
           W_f1, b_f1, W_f2, b_f2, W_f3, b_f3):
    src = edge_index[0].astype(jnp.int32)
    dst = edge_index[1].astype(jnp.int32)

    npad = E_PAD - E
    pad = jnp.arange(npad, dtype=jnp.int32)
    src_edge_b = jnp.concatenate([src, pad % N]).reshape(NW, CPW, CHUNK)
    src_deg_b = jnp.concatenate([src, N + pad % NDUMMY]).reshape(NW, CPW, CHUNK)
    dst_b = jnp.concatenate([dst, N + pad % NDUMMY]).reshape(NW, CPW, CHUNK)

    degs, degd = _deg_pass(src_deg_b, dst_b)

    hh1 = _tc_scale(actor_input.astype(jnp.float32), degs)
    p1 = _edge_pass(hh1, src_edge_b, dst_b)
    hh2 = _tc_layer(p1, hh1, degd, degs, W_g1, b_g1.reshape(1, -1))
    p2 = _edge_pass(hh2, src_edge_b, dst_b)
    hh3 = _tc_layer(p2, hh2, degd, degs, W_g2, b_g2.reshape(1, -1))
    p3 = _edge_pass(hh3, src_edge_b, dst_b)
    prob, gnn_output = _tc_final(p3, hh3, degd, W_g3, b_g3.reshape(1, -1),
                                 W_f1, b_f1.reshape(1, -1),
                                 W_f2, b_f2.reshape(1, -1),
                                 W_f3, b_f3.reshape(1, -1))
    return (prob, gnn_output)


# speedup vs baseline: 12.4841x; 1.1279x over previous
import functools

import jax
import jax.numpy as jnp
from jax import lax
from jax.experimental import pallas as pl
from jax.experimental.pallas import tpu as pltpu
from jax.experimental.pallas import tpu_sc as plsc

N = 10000
E = 320000
D = 128
NW = 32
CHUNK = 128
CPW = 80
EPW = CHUNK * CPW
E_PAD = NW * EPW
NROWS = N + 112
ZROWS = NROWS // 16
NDUMMY = 112

_mesh = plsc.VectorSubcoreMesh(core_axis_name="c", subcore_axis_name="s")


@functools.partial(
    pl.kernel,
    out_type=(
        jax.ShapeDtypeStruct((2, NROWS, 16), jnp.float32),
        jax.ShapeDtypeStruct((2, NROWS, 16), jnp.float32),
    ),
    mesh=_mesh,
    scratch_types=[
        pltpu.VMEM((CPW, CHUNK), jnp.int32),
        pltpu.VMEM((CPW, CHUNK), jnp.int32),
        pltpu.VMEM((CHUNK, 16), jnp.float32),
        pltpu.VMEM_SHARED((NROWS, 16), jnp.float32),
        pltpu.VMEM_SHARED((NROWS, 16), jnp.float32),
    ],
)
def _deg_pass(srcb_hbm, dstb_hbm, degs_hbm, degd_hbm,
              src_v, dst_v, ones_v, degs_sh, degd_sh):
    c = lax.axis_index("c")
    s = lax.axis_index("s")
    w = c * 16 + s

    vzero = jnp.zeros((16,), jnp.float32)
    vone = jnp.ones((16,), jnp.float32)

    def fill(i, _):
        ones_v[i, pl.ds(0, 16)] = vzero
        return 0

    lax.fori_loop(0, CHUNK, fill, 0)

    zb = s * ZROWS
    for tgt in (degs_sh, degd_sh):
        for k in range(4):
            pltpu.sync_copy(ones_v, tgt.at[pl.ds(zb + k * 128, 128)])
        pltpu.sync_copy(ones_v.at[pl.ds(0, ZROWS - 512)],
                        tgt.at[pl.ds(zb + 512, ZROWS - 512)])

    def refill(i, _):
        ones_v[i, pl.ds(0, 16)] = vone
        return 0

    lax.fori_loop(0, CHUNK, refill, 0)

    pltpu.sync_copy(srcb_hbm.at[w], src_v)
    pltpu.sync_copy(dstb_hbm.at[w], dst_v)
    plsc.subcore_barrier()

    def chunk(j, _):
        pltpu.sync_copy(ones_v, degs_sh.at[src_v.at[j]], add=True)
        pltpu.sync_copy(ones_v, degd_sh.at[dst_v.at[j]], add=True)
        return 0

    lax.fori_loop(0, CPW, chunk, 0)
    plsc.subcore_barrier()

    pltpu.sync_copy(degs_sh.at[pl.ds(zb, ZROWS)], degs_hbm.at[c, pl.ds(zb, ZROWS)])
    pltpu.sync_copy(degd_sh.at[pl.ds(zb, ZROWS)], degd_hbm.at[c, pl.ds(zb, ZROWS)])


@functools.partial(
    pl.kernel,
    out_type=jax.ShapeDtypeStruct((2, NROWS, D), jnp.float32),
    mesh=_mesh,
    scratch_types=[
        pltpu.VMEM((CPW // 2, CHUNK), jnp.int32),
        pltpu.VMEM((CPW // 2, CHUNK), jnp.int32),
        pltpu.VMEM((CHUNK, D), jnp.float32),
        pltpu.VMEM((CHUNK, D), jnp.float32),
        pltpu.VMEM_SHARED((NROWS, D), jnp.float32),
        pltpu.SemaphoreType.DMA,
        pltpu.SemaphoreType.DMA,
    ],
)
def _edge_pass(hh_hbm, srcb_hbm, dstb_hbm, out_hbm,
               src_v, dst_v, rows_a, rows_b, agg_sh, sem_a, sem_b):
    c = lax.axis_index("c")
    s = lax.axis_index("s")
    w = c * 16 + s

    vzero = jnp.zeros((16,), jnp.float32)

    def zrow(i, _):
        for k in range(8):
            rows_a[i, pl.ds(k * 16, 16)] = vzero
        return 0

    lax.fori_loop(0, CHUNK, zrow, 0)

    zb = s * ZROWS
    for k in range(4):
        pltpu.sync_copy(rows_a, agg_sh.at[pl.ds(zb + k * 128, 128)])
    pltpu.sync_copy(rows_a.at[pl.ds(0, ZROWS - 512)],
                    agg_sh.at[pl.ds(zb + 512, ZROWS - 512)])

    plsc.subcore_barrier()

    hcpw = CPW // 2
    npair = hcpw // 2

    def pair(t, _):
        j0 = 2 * t
        ga = pltpu.async_copy(hh_hbm.at[src_v.at[j0]], rows_a, sem_a)
        gb = pltpu.async_copy(hh_hbm.at[src_v.at[j0 + 1]], rows_b, sem_b)
        ga.wait()
        pltpu.sync_copy(rows_a, agg_sh.at[dst_v.at[j0]], add=True)
        gb.wait()
        pltpu.sync_copy(rows_b, agg_sh.at[dst_v.at[j0 + 1]], add=True)
        return 0

    for half in range(2):
        pltpu.sync_copy(srcb_hbm.at[w, pl.ds(half * hcpw, hcpw)], src_v)
        pltpu.sync_copy(dstb_hbm.at[w, pl.ds(half * hcpw, hcpw)], dst_v)
        lax.fori_loop(0, npair, pair, 0)

    plsc.subcore_barrier()

    pltpu.sync_copy(agg_sh.at[pl.ds(zb, ZROWS)], out_hbm.at[c, pl.ds(zb, ZROWS)])


_BLK = 1000
_GRID = N // _BLK


def _col0_norm(dref):
    d = dref[0, :, 0:1] + dref[1, :, 0:1] + 1.0
    return lax.rsqrt(d)


def _scale_body(x_ref, ds_ref, o_ref):
    o_ref[...] = x_ref[...] * _col0_norm(ds_ref)


def _layer_body(p_ref, hh_ref, dd_ref, ds_ref, w_ref, b_ref, o_ref):
    agg = (p_ref[0] + p_ref[1] + hh_ref[...]) * _col0_norm(dd_ref)
    h = jnp.dot(agg, w_ref[...], preferred_element_type=jnp.float32) + b_ref[...]
    h = jnp.maximum(h, 0.0)
    o_ref[...] = h * _col0_norm(ds_ref)


def _final_body(p_ref, hh_ref, dd_ref, wg_ref, bg_ref,
                w1_ref, b1_ref, w2_ref, b2_ref, w3_ref, b3_ref,
                prob_ref, g_ref):
    agg = (p_ref[0] + p_ref[1] + hh_ref[...]) * _col0_norm(dd_ref)
    z = jnp.dot(agg, wg_ref[...], preferred_element_type=jnp.float32) + bg_ref[...]
    g = jax.nn.sigmoid(z) + 1e-8
    g_ref[...] = g
    x = jnp.maximum(jnp.dot(g, w1_ref[...], preferred_element_type=jnp.float32)
                    + b1_ref[...], 0.0)
    x = jnp.maximum(jnp.dot(x, w2_ref[...], preferred_element_type=jnp.float32)
                    + b2_ref[...], 0.0)
    lg = jnp.dot(x, w3_ref[...], preferred_element_type=jnp.float32) + b3_ref[...]
    m = jnp.max(lg, axis=-1, keepdims=True)
    e = jnp.exp(lg - m)
    prob_ref[...] = e / jnp.sum(e, axis=-1, keepdims=True)


def _rows_spec(width):
    return pl.BlockSpec((_BLK, width), lambda i: (i, 0))


def _deg_spec():
    return pl.BlockSpec((2, _BLK, 16), lambda i: (0, i, 0))


def _parts_spec():
    return pl.BlockSpec((2, _BLK, D), lambda i: (0, i, 0))


def _w_spec(r, k):
    return pl.BlockSpec((r, k), lambda i: (0, 0))


def _tc_scale(x, degs):
    return pl.pallas_call(
        _scale_body,
        grid=(_GRID,),
        in_specs=[_rows_spec(D), _deg_spec()],
        out_specs=_rows_spec(D),
        out_shape=jax.ShapeDtypeStruct((N, D), jnp.float32),
    )(x, degs)


def _tc_layer(parts, hh, degd, degs, w, b):
    return pl.pallas_call(
        _layer_body,
        grid=(_GRID,),
        in_specs=[_parts_spec(), _rows_spec(D), _deg_spec(), _deg_spec(),
                  _w_spec(D, D), _w_spec(1, D)],
        out_specs=_rows_spec(D),
        out_shape=jax.ShapeDtypeStruct((N, D), jnp.float32),
    )(parts, hh, degd, degs, w, b)


def _tc_final(parts, hh, degd, wg, bg, w1, b1, w2, b2, w3, b3):
    return pl.pallas_call(
        _final_body,
        grid=(_GRID,),
        in_specs=[_parts_spec(), _rows_spec(D), _deg_spec(),
                  _w_spec(D, D), _w_spec(1, D),
                  _w_spec(D, 256), _w_spec(1, 256),
                  _w_spec(256, 256), _w_spec(1, 256),
                  _w_spec(256, 16), _w_spec(1, 16)],
        out_specs=[_rows_spec(16), _rows_spec(D)],
        out_shape=[jax.ShapeDtypeStruct((N, 16), jnp.float32),
                   jax.ShapeDtypeStruct((N, D), jnp.float32)],
    )(parts, hh, degd, wg, bg, w1, b1, w2, b2, w3, b3)


def kernel(actor_input, edge_index, W_g1, b_g1, W_g2, b_g2, W_g3, b_g3,
           W_f1, b_f1, W_f2, b_f2, W_f3, b_f3):
    src = edge_index[0].astype(jnp.int32)
    dst = edge_index[1].astype(jnp.int32)

    npad = E_PAD - E
    pad = jnp.arange(npad, dtype=jnp.int32)
    src_edge_b = jnp.concatenate([src, pad % N]).reshape(NW, CPW, CHUNK)
    src_deg_b = jnp.concatenate([src, N + pad % NDUMMY]).reshape(NW, CPW, CHUNK)
    dst_b = jnp.concatenate([dst, N + pad % NDUMMY]).reshape(NW, CPW, CHUNK)

    degs, degd = _deg_pass(src_deg_b, dst_b)

    hh1 = _tc_scale(actor_input.astype(jnp.float32), degs)
    p1 = _edge_pass(hh1, src_edge_b, dst_b)
    hh2 = _tc_layer(p1, hh1, degd, degs, W_g1, b_g1.reshape(1, -1))
    p2 = _edge_pass(hh2, src_edge_b, dst_b)
    hh3 = _tc_layer(p2, hh2, degd, degs, W_g2, b_g2.reshape(1, -1))
    p3 = _edge_pass(hh3, src_edge_b, dst_b)
    prob, gnn_output = _tc_final(p3, hh3, degd, W_g3, b_g3.reshape(1, -1),
                                 W_f1, b_f1.reshape(1, -1),
                                 W_f2, b_f2.reshape(1, -1),
                                 W_f3, b_f3.reshape(1, -1))
    return (prob, gnn_output)


# speedup vs baseline: 12.7684x; 1.0228x over previous
import functools

import jax
import jax.numpy as jnp
from jax import lax
from jax.experimental import pallas as pl
from jax.experimental.pallas import tpu as pltpu
from jax.experimental.pallas import tpu_sc as plsc

N = 10000
E = 320000
D = 128
NW = 32
CHUNK = 128
CPW = 80
EPW = CHUNK * CPW
E_PAD = NW * EPW
NROWS = N + 112
ZROWS = NROWS // 16
NDUMMY = 112

_mesh = plsc.VectorSubcoreMesh(core_axis_name="c", subcore_axis_name="s")


@functools.partial(
    pl.kernel,
    out_type=(
        jax.ShapeDtypeStruct((2, NROWS, 16), jnp.float32),
        jax.ShapeDtypeStruct((2, NROWS, 16), jnp.float32),
    ),
    mesh=_mesh,
    scratch_types=[
        pltpu.VMEM((CPW, CHUNK), jnp.int32),
        pltpu.VMEM((CPW, CHUNK), jnp.int32),
        pltpu.VMEM((CHUNK, 16), jnp.float32),
        pltpu.VMEM_SHARED((NROWS, 16), jnp.float32),
        pltpu.VMEM_SHARED((NROWS, 16), jnp.float32),
        pltpu.SemaphoreType.DMA,
        pltpu.SemaphoreType.DMA,
    ],
)
def _deg_pass(srcb_hbm, dstb_hbm, degs_hbm, degd_hbm,
              src_v, dst_v, ones_v, degs_sh, degd_sh, sem_a, sem_b):
    c = lax.axis_index("c")
    s = lax.axis_index("s")
    w = c * 16 + s

    vzero = jnp.zeros((16,), jnp.float32)
    vone = jnp.ones((16,), jnp.float32)

    def fill(i, _):
        ones_v[i, pl.ds(0, 16)] = vzero
        return 0

    lax.fori_loop(0, CHUNK, fill, 0)

    zb = s * ZROWS
    for tgt in (degs_sh, degd_sh):
        for k in range(4):
            pltpu.sync_copy(ones_v, tgt.at[pl.ds(zb + k * 128, 128)])
        pltpu.sync_copy(ones_v.at[pl.ds(0, ZROWS - 512)],
                        tgt.at[pl.ds(zb + 512, ZROWS - 512)])

    def refill(i, _):
        ones_v[i, pl.ds(0, 16)] = vone
        return 0

    lax.fori_loop(0, CHUNK, refill, 0)

    pltpu.sync_copy(srcb_hbm.at[w], src_v)
    pltpu.sync_copy(dstb_hbm.at[w], dst_v)
    plsc.subcore_barrier()

    def chunk(j, _):
        da = pltpu.async_copy(ones_v, degs_sh.at[src_v.at[j]], sem_a, add=True)
        db = pltpu.async_copy(ones_v, degd_sh.at[dst_v.at[j]], sem_b, add=True)
        da.wait()
        db.wait()
        return 0

    lax.fori_loop(0, CPW, chunk, 0)
    plsc.subcore_barrier()

    pltpu.sync_copy(degs_sh.at[pl.ds(zb, ZROWS)], degs_hbm.at[c, pl.ds(zb, ZROWS)])
    pltpu.sync_copy(degd_sh.at[pl.ds(zb, ZROWS)], degd_hbm.at[c, pl.ds(zb, ZROWS)])


@functools.partial(
    pl.kernel,
    out_type=jax.ShapeDtypeStruct((2, NROWS, D), jnp.float32),
    mesh=_mesh,
    scratch_types=[
        pltpu.VMEM((CPW // 2, CHUNK), jnp.int32),
        pltpu.VMEM((CPW // 2, CHUNK), jnp.int32),
        pltpu.VMEM((CHUNK, D), jnp.float32),
        pltpu.VMEM((CHUNK, D), jnp.float32),
        pltpu.VMEM_SHARED((NROWS, D), jnp.float32),
        pltpu.SemaphoreType.DMA,
        pltpu.SemaphoreType.DMA,
        pltpu.SemaphoreType.DMA,
        pltpu.SemaphoreType.DMA,
    ],
)
def _edge_pass(hh_hbm, srcb_hbm, dstb_hbm, out_hbm,
               src_v, dst_v, rows_a, rows_b, agg_sh,
               sem_a, sem_b, sem_sa, sem_sb):
    c = lax.axis_index("c")
    s = lax.axis_index("s")
    w = c * 16 + s

    vzero = jnp.zeros((16,), jnp.float32)

    def zrow(i, _):
        for k in range(8):
            rows_a[i, pl.ds(k * 16, 16)] = vzero
        return 0

    lax.fori_loop(0, CHUNK, zrow, 0)

    zb = s * ZROWS
    for k in range(4):
        pltpu.sync_copy(rows_a, agg_sh.at[pl.ds(zb + k * 128, 128)])
    pltpu.sync_copy(rows_a.at[pl.ds(0, ZROWS - 512)],
                    agg_sh.at[pl.ds(zb + 512, ZROWS - 512)])

    plsc.subcore_barrier()

    hcpw = CPW // 2
    npair = hcpw // 2

    def pair(t, _):
        j0 = 2 * t
        ga = pltpu.async_copy(hh_hbm.at[src_v.at[j0]], rows_a, sem_a)
        gb = pltpu.async_copy(hh_hbm.at[src_v.at[j0 + 1]], rows_b, sem_b)
        ga.wait()
        sa = pltpu.async_copy(rows_a, agg_sh.at[dst_v.at[j0]], sem_sa, add=True)
        gb.wait()
        sb = pltpu.async_copy(rows_b, agg_sh.at[dst_v.at[j0 + 1]], sem_sb, add=True)
        sa.wait()
        sb.wait()
        return 0

    for half in range(2):
        pltpu.sync_copy(srcb_hbm.at[w, pl.ds(half * hcpw, hcpw)], src_v)
        pltpu.sync_copy(dstb_hbm.at[w, pl.ds(half * hcpw, hcpw)], dst_v)
        lax.fori_loop(0, npair, pair, 0)

    plsc.subcore_barrier()

    pltpu.sync_copy(agg_sh.at[pl.ds(zb, ZROWS)], out_hbm.at[c, pl.ds(zb, ZROWS)])


_BLK = 1000
_GRID = N // _BLK


def _col0_norm(dref):
    d = dref[0, :, 0:1] + dref[1, :, 0:1] + 1.0
    return lax.rsqrt(d)


def _scale_body(x_ref, ds_ref, o_ref):
    o_ref[...] = x_ref[...] * _col0_norm(ds_ref)


def _layer_body(p_ref, hh_ref, dd_ref, ds_ref, w_ref, b_ref, o_ref):
    agg = (p_ref[0] + p_ref[1] + hh_ref[...]) * _col0_norm(dd_ref)
    h = jnp.dot(agg, w_ref[...], preferred_element_type=jnp.float32) + b_ref[...]
    h = jnp.maximum(h, 0.0)
    o_ref[...] = h * _col0_norm(ds_ref)


def _final_body(p_ref, hh_ref, dd_ref, wg_ref, bg_ref,
                w1_ref, b1_ref, w2_ref, b2_ref, w3_ref, b3_ref,
                prob_ref, g_ref):
    agg = (p_ref[0] + p_ref[1] + hh_ref[...]) * _col0_norm(dd_ref)
    z = jnp.dot(agg, wg_ref[...], preferred_element_type=jnp.float32) + bg_ref[...]
    g = jax.nn.sigmoid(z) + 1e-8
    g_ref[...] = g
    x = jnp.maximum(jnp.dot(g, w1_ref[...], preferred_element_type=jnp.float32)
                    + b1_ref[...], 0.0)
    x = jnp.maximum(jnp.dot(x, w2_ref[...], preferred_element_type=jnp.float32)
                    + b2_ref[...], 0.0)
    lg = jnp.dot(x, w3_ref[...], preferred_element_type=jnp.float32) + b3_ref[...]
    m = jnp.max(lg, axis=-1, keepdims=True)
    e = jnp.exp(lg - m)
    prob_ref[...] = e / jnp.sum(e, axis=-1, keepdims=True)


def _rows_spec(width):
    return pl.BlockSpec((_BLK, width), lambda i: (i, 0))


def _deg_spec():
    return pl.BlockSpec((2, _BLK, 16), lambda i: (0, i, 0))


def _parts_spec():
    return pl.BlockSpec((2, _BLK, D), lambda i: (0, i, 0))


def _w_spec(r, k):
    return pl.BlockSpec((r, k), lambda i: (0, 0))


def _tc_scale(x, degs):
    return pl.pallas_call(
        _scale_body,
        grid=(_GRID,),
        in_specs=[_rows_spec(D), _deg_spec()],
        out_specs=_rows_spec(D),
        out_shape=jax.ShapeDtypeStruct((N, D), jnp.float32),
    )(x, degs)


def _tc_layer(parts, hh, degd, degs, w, b):
    return pl.pallas_call(
        _layer_body,
        grid=(_GRID,),
        in_specs=[_parts_spec(), _rows_spec(D), _deg_spec(), _deg_spec(),
                  _w_spec(D, D), _w_spec(1, D)],
        out_specs=_rows_spec(D),
        out_shape=jax.ShapeDtypeStruct((N, D), jnp.float32),
    )(parts, hh, degd, degs, w, b)


def _tc_final(parts, hh, degd, wg, bg, w1, b1, w2, b2, w3, b3):
    return pl.pallas_call(
        _final_body,
        grid=(_GRID,),
        in_specs=[_parts_spec(), _rows_spec(D), _deg_spec(),
                  _w_spec(D, D), _w_spec(1, D),
                  _w_spec(D, 256), _w_spec(1, 256),
                  _w_spec(256, 256), _w_spec(1, 256),
                  _w_spec(256, 16), _w_spec(1, 16)],
        out_specs=[_rows_spec(16), _rows_spec(D)],
        out_shape=[jax.ShapeDtypeStruct((N, 16), jnp.float32),
                   jax.ShapeDtypeStruct((N, D), jnp.float32)],
    )(parts, hh, degd, wg, bg, w1, b1, w2, b2, w3, b3)


def kernel(actor_input, edge_index, W_g1, b_g1, W_g2, b_g2, W_g3, b_g3,
           W_f1, b_f1, W_f2, b_f2, W_f3, b_f3):
    src = edge_index[0].astype(jnp.int32)
    dst = edge_index[1].astype(jnp.int32)

    npad = E_PAD - E
    pad = jnp.arange(npad, dtype=jnp.int32)
    src_edge_b = jnp.concatenate([src, pad % N]).reshape(NW, CPW, CHUNK)
    src_deg_b = jnp.concatenate([src, N + pad % NDUMMY]).reshape(NW, CPW, CHUNK)
    dst_b = jnp.concatenate([dst, N + pad % NDUMMY]).reshape(NW, CPW, CHUNK)

    degs, degd = _deg_pass(src_deg_b, dst_b)

    hh1 = _tc_scale(actor_input.astype(jnp.float32), degs)
    p1 = _edge_pass(hh1, src_edge_b, dst_b)
    hh2 = _tc_layer(p1, hh1, degd, degs, W_g1, b_g1.reshape(1, -1))
    p2 = _edge_pass(hh2, src_edge_b, dst_b)
    hh3 = _tc_layer(p2, hh2, degd, degs, W_g2, b_g2.reshape(1, -1))
    p3 = _edge_pass(hh3, src_edge_b, dst_b)
    prob, gnn_output = _tc_final(p3, hh3, degd, W_g3, b_g3.reshape(1, -1),
                                 W_f1, b_f1.reshape(1, -1),
                                 W_f2, b_f2.reshape(1, -1),
                                 W_f3, b_f3.reshape(1, -1))
    return (prob, gnn_output)


# speedup vs baseline: 12.9415x; 1.0136x over previous
import functools

import jax
import jax.numpy as jnp
from jax import lax
from jax.experimental import pallas as pl
from jax.experimental.pallas import tpu as pltpu
from jax.experimental.pallas import tpu_sc as plsc

N = 10000
E = 320000
D = 128
NW = 32
CHUNK = 128
CPW = 80
EPW = CHUNK * CPW
E_PAD = NW * EPW
NROWS = N + 112
ZROWS = NROWS // 16
NDUMMY = 112

_mesh = plsc.VectorSubcoreMesh(core_axis_name="c", subcore_axis_name="s")


@functools.partial(
    pl.kernel,
    out_type=(
        jax.ShapeDtypeStruct((2, NROWS, 16), jnp.float32),
        jax.ShapeDtypeStruct((2, NROWS, 16), jnp.float32),
    ),
    mesh=_mesh,
    scratch_types=[
        pltpu.VMEM((CPW, CHUNK), jnp.int32),
        pltpu.VMEM((CPW, CHUNK), jnp.int32),
        pltpu.VMEM((CHUNK, 16), jnp.float32),
        pltpu.VMEM_SHARED((NROWS, 16), jnp.float32),
        pltpu.VMEM_SHARED((NROWS, 16), jnp.float32),
        pltpu.SemaphoreType.DMA,
        pltpu.SemaphoreType.DMA,
    ],
)
def _deg_pass(srcb_hbm, dstb_hbm, degs_hbm, degd_hbm,
              src_v, dst_v, ones_v, degs_sh, degd_sh, sem_a, sem_b):
    c = lax.axis_index("c")
    s = lax.axis_index("s")
    w = c * 16 + s

    vzero = jnp.zeros((16,), jnp.float32)
    vone = jnp.ones((16,), jnp.float32)

    def fill(i, _):
        ones_v[i, pl.ds(0, 16)] = vzero
        return 0

    lax.fori_loop(0, CHUNK, fill, 0)

    zb = s * ZROWS
    for tgt in (degs_sh, degd_sh):
        for k in range(4):
            pltpu.sync_copy(ones_v, tgt.at[pl.ds(zb + k * 128, 128)])
        pltpu.sync_copy(ones_v.at[pl.ds(0, ZROWS - 512)],
                        tgt.at[pl.ds(zb + 512, ZROWS - 512)])

    def refill(i, _):
        ones_v[i, pl.ds(0, 16)] = vone
        return 0

    lax.fori_loop(0, CHUNK, refill, 0)

    pltpu.sync_copy(srcb_hbm.at[w], src_v)
    pltpu.sync_copy(dstb_hbm.at[w], dst_v)
    plsc.subcore_barrier()

    def chunk(j, _):
        da = pltpu.async_copy(ones_v, degs_sh.at[src_v.at[j]], sem_a, add=True)
        db = pltpu.async_copy(ones_v, degd_sh.at[dst_v.at[j]], sem_b, add=True)
        da.wait()
        db.wait()
        return 0

    lax.fori_loop(0, CPW, chunk, 0)
    plsc.subcore_barrier()

    pltpu.sync_copy(degs_sh.at[pl.ds(zb, ZROWS)], degs_hbm.at[c, pl.ds(zb, ZROWS)])
    pltpu.sync_copy(degd_sh.at[pl.ds(zb, ZROWS)], degd_hbm.at[c, pl.ds(zb, ZROWS)])


@functools.partial(
    pl.kernel,
    out_type=jax.ShapeDtypeStruct((2, NROWS, D), jnp.float32),
    mesh=_mesh,
    scratch_types=[
        pltpu.VMEM((CPW // 2, CHUNK), jnp.int32),
        pltpu.VMEM((CPW // 2, CHUNK), jnp.int32),
        pltpu.VMEM((CHUNK, D), jnp.float32),
        pltpu.VMEM((CHUNK, D), jnp.float32),
        pltpu.VMEM_SHARED((NROWS, D), jnp.float32),
        pltpu.SemaphoreType.DMA,
        pltpu.SemaphoreType.DMA,
        pltpu.SemaphoreType.DMA,
        pltpu.SemaphoreType.DMA,
    ],
)
def _edge_pass(hh_hbm, srcb_hbm, dstb_hbm, out_hbm,
               src_v, dst_v, rows_a, rows_b, agg_sh,
               sem_a, sem_b, sem_sa, sem_sb):
    c = lax.axis_index("c")
    s = lax.axis_index("s")
    w = c * 16 + s

    vzero = jnp.zeros((16,), jnp.float32)

    def zrow(i, _):
        for k in range(8):
            rows_a[i, pl.ds(k * 16, 16)] = vzero
        return 0

    lax.fori_loop(0, CHUNK, zrow, 0)

    zb = s * ZROWS
    for k in range(4):
        pltpu.sync_copy(rows_a, agg_sh.at[pl.ds(zb + k * 128, 128)])
    pltpu.sync_copy(rows_a.at[pl.ds(0, ZROWS - 512)],
                    agg_sh.at[pl.ds(zb + 512, ZROWS - 512)])

    plsc.subcore_barrier()

    hcpw = CPW // 2
    npair = hcpw // 2

    def pair(t, _):
        j0 = 2 * t
        pltpu.make_async_copy(hh_hbm.at[src_v.at[j0]], rows_a, sem_a).wait()
        sa = pltpu.async_copy(rows_a, agg_sh.at[dst_v.at[j0]], sem_sa, add=True)
        pltpu.make_async_copy(hh_hbm.at[src_v.at[j0 + 1]], rows_b, sem_b).wait()
        sb = pltpu.async_copy(rows_b, agg_sh.at[dst_v.at[j0 + 1]], sem_sb, add=True)
        sa.wait()
        pltpu.async_copy(hh_hbm.at[src_v.at[j0 + 2]], rows_a, sem_a)
        sb.wait()
        pltpu.async_copy(hh_hbm.at[src_v.at[j0 + 3]], rows_b, sem_b)
        return 0

    for half in range(2):
        pltpu.sync_copy(srcb_hbm.at[w, pl.ds(half * hcpw, hcpw)], src_v)
        pltpu.sync_copy(dstb_hbm.at[w, pl.ds(half * hcpw, hcpw)], dst_v)
        pltpu.async_copy(hh_hbm.at[src_v.at[0]], rows_a, sem_a)
        pltpu.async_copy(hh_hbm.at[src_v.at[1]], rows_b, sem_b)
        lax.fori_loop(0, npair - 1, pair, 0)
        jl = hcpw - 2
        pltpu.make_async_copy(hh_hbm.at[src_v.at[jl]], rows_a, sem_a).wait()
        pltpu.sync_copy(rows_a, agg_sh.at[dst_v.at[jl]], add=True)
        pltpu.make_async_copy(hh_hbm.at[src_v.at[jl + 1]], rows_b, sem_b).wait()
        pltpu.sync_copy(rows_b, agg_sh.at[dst_v.at[jl + 1]], add=True)

    plsc.subcore_barrier()

    pltpu.sync_copy(agg_sh.at[pl.ds(zb, ZROWS)], out_hbm.at[c, pl.ds(zb, ZROWS)])


_BLK = 1000
_GRID = N // _BLK


def _col0_norm(dref):
    d = dref[0, :, 0:1] + dref[1, :, 0:1] + 1.0
    return lax.rsqrt(d)


def _scale_body(x_ref, ds_ref, o_ref):
    o_ref[...] = x_ref[...] * _col0_norm(ds_ref)


def _layer_body(p_ref, hh_ref, dd_ref, ds_ref, w_ref, b_ref, o_ref):
    agg = (p_ref[0] + p_ref[1] + hh_ref[...]) * _col0_norm(dd_ref)
    h = jnp.dot(agg, w_ref[...], preferred_element_type=jnp.float32) + b_ref[...]
    h = jnp.maximum(h, 0.0)
    o_ref[...] = h * _col0_norm(ds_ref)


def _final_body(p_ref, hh_ref, dd_ref, wg_ref, bg_ref,
                w1_ref, b1_ref, w2_ref, b2_ref, w3_ref, b3_ref,
                prob_ref, g_ref):
    agg = (p_ref[0] + p_ref[1] + hh_ref[...]) * _col0_norm(dd_ref)
    z = jnp.dot(agg, wg_ref[...], preferred_element_type=jnp.float32) + bg_ref[...]
    g = jax.nn.sigmoid(z) + 1e-8
    g_ref[...] = g
    x = jnp.maximum(jnp.dot(g, w1_ref[...], preferred_element_type=jnp.float32)
                    + b1_ref[...], 0.0)
    x = jnp.maximum(jnp.dot(x, w2_ref[...], preferred_element_type=jnp.float32)
                    + b2_ref[...], 0.0)
    lg = jnp.dot(x, w3_ref[...], preferred_element_type=jnp.float32) + b3_ref[...]
    m = jnp.max(lg, axis=-1, keepdims=True)
    e = jnp.exp(lg - m)
    prob_ref[...] = e / jnp.sum(e, axis=-1, keepdims=True)


def _rows_spec(width):
    return pl.BlockSpec((_BLK, width), lambda i: (i, 0))


def _deg_spec():
    return pl.BlockSpec((2, _BLK, 16), lambda i: (0, i, 0))


def _parts_spec():
    return pl.BlockSpec((2, _BLK, D), lambda i: (0, i, 0))


def _w_spec(r, k):
    return pl.BlockSpec((r, k), lambda i: (0, 0))


def _tc_scale(x, degs):
    return pl.pallas_call(
        _scale_body,
        grid=(_GRID,),
        in_specs=[_rows_spec(D), _deg_spec()],
        out_specs=_rows_spec(D),
        out_shape=jax.ShapeDtypeStruct((N, D), jnp.float32),
    )(x, degs)


def _tc_layer(parts, hh, degd, degs, w, b):
    return pl.pallas_call(
        _layer_body,
        grid=(_GRID,),
        in_specs=[_parts_spec(), _rows_spec(D), _deg_spec(), _deg_spec(),
                  _w_spec(D, D), _w_spec(1, D)],
        out_specs=_rows_spec(D),
        out_shape=jax.ShapeDtypeStruct((N, D), jnp.float32),
    )(parts, hh, degd, degs, w, b)


def _tc_final(parts, hh, degd, wg, bg, w1, b1, w2, b2, w3, b3):
    return pl.pallas_call(
        _final_body,
        grid=(_GRID,),
        in_specs=[_parts_spec(), _rows_spec(D), _deg_spec(),
                  _w_spec(D, D), _w_spec(1, D),
                  _w_spec(D, 256), _w_spec(1, 256),
                  _w_spec(256, 256), _w_spec(1, 256),
                  _w_spec(256, 16), _w_spec(1, 16)],
        out_specs=[_rows_spec(16), _rows_spec(D)],
        out_shape=[jax.ShapeDtypeStruct((N, 16), jnp.float32),
                   jax.ShapeDtypeStruct((N, D), jnp.float32)],
    )(parts, hh, degd, wg, bg, w1, b1, w2, b2, w3, b3)


def kernel(actor_input, edge_index, W_g1, b_g1, W_g2, b_g2, W_g3, b_g3,
           W_f1, b_f1, W_f2, b_f2, W_f3, b_f3):
    src = edge_index[0].astype(jnp.int32)
    dst = edge_index[1].astype(jnp.int32)

    npad = E_PAD - E
    pad = jnp.arange(npad, dtype=jnp.int32)
    src_edge_b = jnp.concatenate([src, pad % N]).reshape(NW, CPW, CHUNK)
    src_deg_b = jnp.concatenate([src, N + pad % NDUMMY]).reshape(NW, CPW, CHUNK)
    dst_b = jnp.concatenate([dst, N + pad % NDUMMY]).reshape(NW, CPW, CHUNK)

    degs, degd = _deg_pass(src_deg_b, dst_b)

    hh1 = _tc_scale(actor_input.astype(jnp.float32), degs)
    p1 = _edge_pass(hh1, src_edge_b, dst_b)
    hh2 = _tc_layer(p1, hh1, degd, degs, W_g1, b_g1.reshape(1, -1))
    p2 = _edge_pass(hh2, src_edge_b, dst_b)
    hh3 = _tc_layer(p2, hh2, degd, degs, W_g2, b_g2.reshape(1, -1))
    p3 = _edge_pass(hh3, src_edge_b, dst_b)
    prob, gnn_output = _tc_final(p3, hh3, degd, W_g3, b_g3.reshape(1, -1),
                                 W_f1, b_f1.reshape(1, -1),
                                 W_f2, b_f2.reshape(1, -1),
                                 W_f3, b_f3.reshape(1, -1))
    return (prob, gnn_output)


# speedup vs baseline: 13.1191x; 1.0137x over previous
import functools

import jax
import jax.numpy as jnp
from jax import lax
from jax.experimental import pallas as pl
from jax.experimental.pallas import tpu as pltpu
from jax.experimental.pallas import tpu_sc as plsc

N = 10000
E = 320000
D = 128
NW = 32
CHUNK = 128
CPW = 80
EPW = CHUNK * CPW
E_PAD = NW * EPW
NROWS = N + 112
ZROWS = NROWS // 16
NDUMMY = 112

_mesh = plsc.VectorSubcoreMesh(core_axis_name="c", subcore_axis_name="s")


@functools.partial(
    pl.kernel,
    out_type=(
        jax.ShapeDtypeStruct((2, NROWS, 16), jnp.float32),
        jax.ShapeDtypeStruct((2, NROWS, 16), jnp.float32),
    ),
    mesh=_mesh,
    scratch_types=[
        pltpu.VMEM((CPW, CHUNK), jnp.int32),
        pltpu.VMEM((CPW, CHUNK), jnp.int32),
        pltpu.VMEM((CHUNK, 16), jnp.float32),
        pltpu.VMEM_SHARED((NROWS, 16), jnp.float32),
        pltpu.VMEM_SHARED((NROWS, 16), jnp.float32),
        pltpu.SemaphoreType.DMA,
        pltpu.SemaphoreType.DMA,
        pltpu.SemaphoreType.DMA,
        pltpu.SemaphoreType.DMA,
    ],
)
def _deg_pass(srcb_hbm, dstb_hbm, degs_hbm, degd_hbm,
              src_v, dst_v, ones_v, degs_sh, degd_sh,
              sem_a, sem_b, sem_c, sem_d):
    c = lax.axis_index("c")
    s = lax.axis_index("s")
    w = c * 16 + s

    vzero = jnp.zeros((16,), jnp.float32)
    vone = jnp.ones((16,), jnp.float32)

    def fill(i, _):
        ones_v[i, pl.ds(0, 16)] = vzero
        return 0

    lax.fori_loop(0, CHUNK, fill, 0)

    zb = s * ZROWS
    for tgt in (degs_sh, degd_sh):
        for k in range(4):
            pltpu.sync_copy(ones_v, tgt.at[pl.ds(zb + k * 128, 128)])
        pltpu.sync_copy(ones_v.at[pl.ds(0, ZROWS - 512)],
                        tgt.at[pl.ds(zb + 512, ZROWS - 512)])

    def refill(i, _):
        ones_v[i, pl.ds(0, 16)] = vone
        return 0

    lax.fori_loop(0, CHUNK, refill, 0)

    pltpu.sync_copy(srcb_hbm.at[w], src_v)
    pltpu.sync_copy(dstb_hbm.at[w], dst_v)
    plsc.subcore_barrier()

    pltpu.async_copy(ones_v, degs_sh.at[src_v.at[0]], sem_a, add=True)
    pltpu.async_copy(ones_v, degd_sh.at[dst_v.at[0]], sem_b, add=True)
    pltpu.async_copy(ones_v, degs_sh.at[src_v.at[1]], sem_c, add=True)
    pltpu.async_copy(ones_v, degd_sh.at[dst_v.at[1]], sem_d, add=True)

    def chunk(t, _):
        j0 = 2 * t
        pltpu.make_async_copy(ones_v, degs_sh.at[src_v.at[j0]], sem_a).wait()
        pltpu.make_async_copy(ones_v, degd_sh.at[dst_v.at[j0]], sem_b).wait()
        pltpu.async_copy(ones_v, degs_sh.at[src_v.at[j0 + 2]], sem_a, add=True)
        pltpu.async_copy(ones_v, degd_sh.at[dst_v.at[j0 + 2]], sem_b, add=True)
        pltpu.make_async_copy(ones_v, degs_sh.at[src_v.at[j0 + 1]], sem_c).wait()
        pltpu.make_async_copy(ones_v, degd_sh.at[dst_v.at[j0 + 1]], sem_d).wait()
        pltpu.async_copy(ones_v, degs_sh.at[src_v.at[j0 + 3]], sem_c, add=True)
        pltpu.async_copy(ones_v, degd_sh.at[dst_v.at[j0 + 3]], sem_d, add=True)
        return 0

    lax.fori_loop(0, CPW // 2 - 1, chunk, 0)
    jl = CPW - 2
    pltpu.make_async_copy(ones_v, degs_sh.at[src_v.at[jl]], sem_a).wait()
    pltpu.make_async_copy(ones_v, degd_sh.at[dst_v.at[jl]], sem_b).wait()
    pltpu.make_async_copy(ones_v, degs_sh.at[src_v.at[jl + 1]], sem_c).wait()
    pltpu.make_async_copy(ones_v, degd_sh.at[dst_v.at[jl + 1]], sem_d).wait()
    plsc.subcore_barrier()

    pltpu.sync_copy(degs_sh.at[pl.ds(zb, ZROWS)], degs_hbm.at[c, pl.ds(zb, ZROWS)])
    pltpu.sync_copy(degd_sh.at[pl.ds(zb, ZROWS)], degd_hbm.at[c, pl.ds(zb, ZROWS)])


@functools.partial(
    pl.kernel,
    out_type=jax.ShapeDtypeStruct((2, NROWS, D), jnp.float32),
    mesh=_mesh,
    scratch_types=[
        pltpu.VMEM((CPW // 2, CHUNK), jnp.int32),
        pltpu.VMEM((CPW // 2, CHUNK), jnp.int32),
        pltpu.VMEM((CHUNK, D), jnp.float32),
        pltpu.VMEM((CHUNK, D), jnp.float32),
        pltpu.VMEM_SHARED((NROWS, D), jnp.float32),
        pltpu.SemaphoreType.DMA,
        pltpu.SemaphoreType.DMA,
        pltpu.SemaphoreType.DMA,
        pltpu.SemaphoreType.DMA,
    ],
)
def _edge_pass(hh_hbm, srcb_hbm, dstb_hbm, out_hbm,
               src_v, dst_v, rows_a, rows_b, agg_sh,
               sem_a, sem_b, sem_sa, sem_sb):
    c = lax.axis_index("c")
    s = lax.axis_index("s")
    w = c * 16 + s

    hcpw = CPW // 2
    npair = hcpw // 2

    ia = pltpu.async_copy(srcb_hbm.at[w, pl.ds(0, hcpw)], src_v, sem_sa)
    ib = pltpu.async_copy(dstb_hbm.at[w, pl.ds(0, hcpw)], dst_v, sem_sb)

    vzero = jnp.zeros((16,), jnp.float32)

    def zrow(i, _):
        for k in range(8):
            rows_a[i, pl.ds(k * 16, 16)] = vzero
        return 0

    lax.fori_loop(0, CHUNK, zrow, 0)

    zb = s * ZROWS
    for k in range(4):
        pltpu.sync_copy(rows_a, agg_sh.at[pl.ds(zb + k * 128, 128)])
    pltpu.sync_copy(rows_a.at[pl.ds(0, ZROWS - 512)],
                    agg_sh.at[pl.ds(zb + 512, ZROWS - 512)])

    ia.wait()
    ib.wait()
    pltpu.async_copy(hh_hbm.at[src_v.at[0]], rows_a, sem_a)
    pltpu.async_copy(hh_hbm.at[src_v.at[1]], rows_b, sem_b)
    plsc.subcore_barrier()


    def pair(t, _):
        j0 = 2 * t
        pltpu.make_async_copy(hh_hbm.at[src_v.at[j0]], rows_a, sem_a).wait()
        sa = pltpu.async_copy(rows_a, agg_sh.at[dst_v.at[j0]], sem_sa, add=True)
        pltpu.make_async_copy(hh_hbm.at[src_v.at[j0 + 1]], rows_b, sem_b).wait()
        sb = pltpu.async_copy(rows_b, agg_sh.at[dst_v.at[j0 + 1]], sem_sb, add=True)
        sa.wait()
        pltpu.async_copy(hh_hbm.at[src_v.at[j0 + 2]], rows_a, sem_a)
        sb.wait()
        pltpu.async_copy(hh_hbm.at[src_v.at[j0 + 3]], rows_b, sem_b)
        return 0

    for half in range(2):
        if half == 1:
            pltpu.sync_copy(srcb_hbm.at[w, pl.ds(hcpw, hcpw)], src_v)
            pltpu.sync_copy(dstb_hbm.at[w, pl.ds(hcpw, hcpw)], dst_v)
            pltpu.async_copy(hh_hbm.at[src_v.at[0]], rows_a, sem_a)
            pltpu.async_copy(hh_hbm.at[src_v.at[1]], rows_b, sem_b)
        lax.fori_loop(0, npair - 1, pair, 0)
        jl = hcpw - 2
        pltpu.make_async_copy(hh_hbm.at[src_v.at[jl]], rows_a, sem_a).wait()
        pltpu.sync_copy(rows_a, agg_sh.at[dst_v.at[jl]], add=True)
        pltpu.make_async_copy(hh_hbm.at[src_v.at[jl + 1]], rows_b, sem_b).wait()
        pltpu.sync_copy(rows_b, agg_sh.at[dst_v.at[jl + 1]], add=True)

    plsc.subcore_barrier()

    pltpu.sync_copy(agg_sh.at[pl.ds(zb, ZROWS)], out_hbm.at[c, pl.ds(zb, ZROWS)])


_BLK = 1000
_GRID = N // _BLK


def _col0_norm(dref):
    d = dref[0, :, 0:1] + dref[1, :, 0:1] + 1.0
    return lax.rsqrt(d)


def _scale_body(x_ref, ds_ref, o_ref):
    o_ref[...] = x_ref[...] * _col0_norm(ds_ref)


def _layer_body(p_ref, hh_ref, dd_ref, ds_ref, w_ref, b_ref, o_ref):
    agg = (p_ref[0] + p_ref[1] + hh_ref[...]) * _col0_norm(dd_ref)
    h = jnp.dot(agg, w_ref[...], preferred_element_type=jnp.float32) + b_ref[...]
    h = jnp.maximum(h, 0.0)
    o_ref[...] = h * _col0_norm(ds_ref)


def _final_body(p_ref, hh_ref, dd_ref, wg_ref, bg_ref,
                w1_ref, b1_ref, w2_ref, b2_ref, w3_ref, b3_ref,
                prob_ref, g_ref):
    agg = (p_ref[0] + p_ref[1] + hh_ref[...]) * _col0_norm(dd_ref)
    z = jnp.dot(agg, wg_ref[...], preferred_element_type=jnp.float32) + bg_ref[...]
    g = jax.nn.sigmoid(z) + 1e-8
    g_ref[...] = g
    x = jnp.maximum(jnp.dot(g, w1_ref[...], preferred_element_type=jnp.float32)
                    + b1_ref[...], 0.0)
    x = jnp.maximum(jnp.dot(x, w2_ref[...], preferred_element_type=jnp.float32)
                    + b2_ref[...], 0.0)
    lg = jnp.dot(x, w3_ref[...], preferred_element_type=jnp.float32) + b3_ref[...]
    m = jnp.max(lg, axis=-1, keepdims=True)
    e = jnp.exp(lg - m)
    prob_ref[...] = e / jnp.sum(e, axis=-1, keepdims=True)


def _rows_spec(width):
    return pl.BlockSpec((_BLK, width), lambda i: (i, 0))


def _deg_spec():
    return pl.BlockSpec((2, _BLK, 16), lambda i: (0, i, 0))


def _parts_spec():
    return pl.BlockSpec((2, _BLK, D), lambda i: (0, i, 0))


def _w_spec(r, k):
    return pl.BlockSpec((r, k), lambda i: (0, 0))


def _tc_scale(x, degs):
    return pl.pallas_call(
        _scale_body,
        grid=(_GRID,),
        in_specs=[_rows_spec(D), _deg_spec()],
        out_specs=_rows_spec(D),
        out_shape=jax.ShapeDtypeStruct((N, D), jnp.float32),
    )(x, degs)


def _tc_layer(parts, hh, degd, degs, w, b):
    return pl.pallas_call(
        _layer_body,
        grid=(_GRID,),
        in_specs=[_parts_spec(), _rows_spec(D), _deg_spec(), _deg_spec(),
                  _w_spec(D, D), _w_spec(1, D)],
        out_specs=_rows_spec(D),
        out_shape=jax.ShapeDtypeStruct((N, D), jnp.float32),
    )(parts, hh, degd, degs, w, b)


def _tc_final(parts, hh, degd, wg, bg, w1, b1, w2, b2, w3, b3):
    return pl.pallas_call(
        _final_body,
        grid=(_GRID,),
        in_specs=[_parts_spec(), _rows_spec(D), _deg_spec(),
                  _w_spec(D, D), _w_spec(1, D),
                  _w_spec(D, 256), _w_spec(1, 256),
                  _w_spec(256, 256), _w_spec(1, 256),
                  _w_spec(256, 16), _w_spec(1, 16)],
        out_specs=[_rows_spec(16), _rows_spec(D)],
        out_shape=[jax.ShapeDtypeStruct((N, 16), jnp.float32),
                   jax.ShapeDtypeStruct((N, D), jnp.float32)],
    )(parts, hh, degd, wg, bg, w1, b1, w2, b2, w3, b3)


def kernel(actor_input, edge_index, W_g1, b_g1, W_g2, b_g2, W_g3, b_g3,
           W_f1, b_f1, W_f2, b_f2, W_f3, b_f3):
    src = edge_index[0].astype(jnp.int32)
    dst = edge_index[1].astype(jnp.int32)

    npad = E_PAD - E
    pad = jnp.arange(npad, dtype=jnp.int32)
    src_edge_b = jnp.concatenate([src, pad % N]).reshape(NW, CPW, CHUNK)
    src_deg_b = jnp.concatenate([src, N + pad % NDUMMY]).reshape(NW, CPW, CHUNK)
    dst_b = jnp.concatenate([dst, N + pad % NDUMMY]).reshape(NW, CPW, CHUNK)

    degs, degd = _deg_pass(src_deg_b, dst_b)

    hh1 = _tc_scale(actor_input.astype(jnp.float32), degs)
    p1 = _edge_pass(hh1, src_edge_b, dst_b)
    hh2 = _tc_layer(p1, hh1, degd, degs, W_g1, b_g1.reshape(1, -1))
    p2 = _edge_pass(hh2, src_edge_b, dst_b)
    hh3 = _tc_layer(p2, hh2, degd, degs, W_g2, b_g2.reshape(1, -1))
    p3 = _edge_pass(hh3, src_edge_b, dst_b)
    prob, gnn_output = _tc_final(p3, hh3, degd, W_g3, b_g3.reshape(1, -1),
                                 W_f1, b_f1.reshape(1, -1),
                                 W_f2, b_f2.reshape(1, -1),
                                 W_f3, b_f3.reshape(1, -1))
    return (prob, gnn_output)


# speedup vs baseline: 13.1303x; 1.0008x over previous
import functools

import jax
import jax.numpy as jnp
from jax import lax
from jax.experimental import pallas as pl
from jax.experimental.pallas import tpu as pltpu
from jax.experimental.pallas import tpu_sc as plsc

N = 10000
E = 320000
D = 128
NW = 32
CHUNK = 128
CPW = 80
EPW = CHUNK * CPW
E_PAD = NW * EPW
NROWS = N + 112
ZROWS = NROWS // 16
NDUMMY = 112

_mesh = plsc.VectorSubcoreMesh(core_axis_name="c", subcore_axis_name="s")


@functools.partial(
    pl.kernel,
    out_type=(
        jax.ShapeDtypeStruct((2, NROWS, 16), jnp.float32),
        jax.ShapeDtypeStruct((2, NROWS, 16), jnp.float32),
    ),
    mesh=_mesh,
    scratch_types=[
        pltpu.VMEM((CPW, CHUNK), jnp.int32),
        pltpu.VMEM((CPW, CHUNK), jnp.int32),
        pltpu.VMEM((CHUNK, 16), jnp.float32),
        pltpu.VMEM_SHARED((NROWS, 16), jnp.float32),
        pltpu.VMEM_SHARED((NROWS, 16), jnp.float32),
        pltpu.SemaphoreType.DMA,
        pltpu.SemaphoreType.DMA,
        pltpu.SemaphoreType.DMA,
        pltpu.SemaphoreType.DMA,
    ],
)
def _deg_pass(srcb_hbm, dstb_hbm, degs_hbm, degd_hbm,
              src_v, dst_v, ones_v, degs_sh, degd_sh,
              sem_a, sem_b, sem_c, sem_d):
    c = lax.axis_index("c")
    s = lax.axis_index("s")
    w = c * 16 + s

    vzero = jnp.zeros((16,), jnp.float32)
    vone = jnp.ones((16,), jnp.float32)

    def fill(i, _):
        ones_v[i, pl.ds(0, 16)] = vzero
        return 0

    lax.fori_loop(0, CHUNK, fill, 0)

    zb = s * ZROWS
    for tgt in (degs_sh, degd_sh):
        for k in range(4):
            pltpu.sync_copy(ones_v, tgt.at[pl.ds(zb + k * 128, 128)])
        pltpu.sync_copy(ones_v.at[pl.ds(0, ZROWS - 512)],
                        tgt.at[pl.ds(zb + 512, ZROWS - 512)])

    def refill(i, _):
        ones_v[i, pl.ds(0, 16)] = vone
        return 0

    lax.fori_loop(0, CHUNK, refill, 0)

    pltpu.sync_copy(srcb_hbm.at[w], src_v)
    pltpu.sync_copy(dstb_hbm.at[w], dst_v)
    plsc.subcore_barrier()

    def chunk(t, _):
        j0 = 2 * t
        d1 = pltpu.async_copy(ones_v, degs_sh.at[src_v.at[j0]], sem_a, add=True)
        d2 = pltpu.async_copy(ones_v, degd_sh.at[dst_v.at[j0]], sem_b, add=True)
        d3 = pltpu.async_copy(ones_v, degs_sh.at[src_v.at[j0 + 1]], sem_c, add=True)
        d4 = pltpu.async_copy(ones_v, degd_sh.at[dst_v.at[j0 + 1]], sem_d, add=True)
        d1.wait()
        d2.wait()
        d3.wait()
        d4.wait()
        return 0

    lax.fori_loop(0, CPW // 2, chunk, 0)
    plsc.subcore_barrier()

    pltpu.sync_copy(degs_sh.at[pl.ds(zb, ZROWS)], degs_hbm.at[c, pl.ds(zb, ZROWS)])
    pltpu.sync_copy(degd_sh.at[pl.ds(zb, ZROWS)], degd_hbm.at[c, pl.ds(zb, ZROWS)])


@functools.partial(
    pl.kernel,
    out_type=jax.ShapeDtypeStruct((2, NROWS, D), jnp.float32),
    mesh=_mesh,
    scratch_types=[
        pltpu.VMEM((CPW // 2, CHUNK), jnp.int32),
        pltpu.VMEM((CPW // 2, CHUNK), jnp.int32),
        pltpu.VMEM((CHUNK, D), jnp.float32),
        pltpu.VMEM((CHUNK, D), jnp.float32),
        pltpu.VMEM_SHARED((NROWS, D), jnp.float32),
        pltpu.SemaphoreType.DMA,
        pltpu.SemaphoreType.DMA,
        pltpu.SemaphoreType.DMA,
        pltpu.SemaphoreType.DMA,
    ],
)
def _edge_pass(hh_hbm, srcb_hbm, dstb_hbm, out_hbm,
               src_v, dst_v, rows_a, rows_b, agg_sh,
               sem_a, sem_b, sem_sa, sem_sb):
    c = lax.axis_index("c")
    s = lax.axis_index("s")
    w = c * 16 + s

    hcpw = CPW // 2
    npair = hcpw // 2

    ia = pltpu.async_copy(srcb_hbm.at[w, pl.ds(0, hcpw)], src_v, sem_sa)
    ib = pltpu.async_copy(dstb_hbm.at[w, pl.ds(0, hcpw)], dst_v, sem_sb)

    vzero = jnp.zeros((16,), jnp.float32)

    def zrow(i, _):
        for k in range(8):
            rows_a[i, pl.ds(k * 16, 16)] = vzero
        return 0

    lax.fori_loop(0, CHUNK, zrow, 0)

    zb = s * ZROWS
    for k in range(4):
        pltpu.sync_copy(rows_a, agg_sh.at[pl.ds(zb + k * 128, 128)])
    pltpu.sync_copy(rows_a.at[pl.ds(0, ZROWS - 512)],
                    agg_sh.at[pl.ds(zb + 512, ZROWS - 512)])

    ia.wait()
    ib.wait()
    pltpu.async_copy(hh_hbm.at[src_v.at[0]], rows_a, sem_a)
    pltpu.async_copy(hh_hbm.at[src_v.at[1]], rows_b, sem_b)
    plsc.subcore_barrier()


    def pair(t, _):
        j0 = 2 * t
        pltpu.make_async_copy(hh_hbm.at[src_v.at[j0]], rows_a, sem_a).wait()
        sa = pltpu.async_copy(rows_a, agg_sh.at[dst_v.at[j0]], sem_sa, add=True)
        pltpu.make_async_copy(hh_hbm.at[src_v.at[j0 + 1]], rows_b, sem_b).wait()
        sb = pltpu.async_copy(rows_b, agg_sh.at[dst_v.at[j0 + 1]], sem_sb, add=True)
        sa.wait()
        pltpu.async_copy(hh_hbm.at[src_v.at[j0 + 2]], rows_a, sem_a)
        sb.wait()
        pltpu.async_copy(hh_hbm.at[src_v.at[j0 + 3]], rows_b, sem_b)
        return 0

    for half in range(2):
        if half == 1:
            pltpu.sync_copy(srcb_hbm.at[w, pl.ds(hcpw, hcpw)], src_v)
            pltpu.sync_copy(dstb_hbm.at[w, pl.ds(hcpw, hcpw)], dst_v)
            pltpu.async_copy(hh_hbm.at[src_v.at[0]], rows_a, sem_a)
            pltpu.async_copy(hh_hbm.at[src_v.at[1]], rows_b, sem_b)
        lax.fori_loop(0, npair - 1, pair, 0)
        jl = hcpw - 2
        pltpu.make_async_copy(hh_hbm.at[src_v.at[jl]], rows_a, sem_a).wait()
        pltpu.sync_copy(rows_a, agg_sh.at[dst_v.at[jl]], add=True)
        pltpu.make_async_copy(hh_hbm.at[src_v.at[jl + 1]], rows_b, sem_b).wait()
        pltpu.sync_copy(rows_b, agg_sh.at[dst_v.at[jl + 1]], add=True)

    plsc.subcore_barrier()

    pltpu.sync_copy(agg_sh.at[pl.ds(zb, ZROWS)], out_hbm.at[c, pl.ds(zb, ZROWS)])


_BLK = 1000
_GRID = N // _BLK


def _col0_norm(dref):
    d = dref[0, :, 0:1] + dref[1, :, 0:1] + 1.0
    return lax.rsqrt(d)


def _scale_body(x_ref, ds_ref, o_ref):
    o_ref[...] = x_ref[...] * _col0_norm(ds_ref)


def _layer_body(p_ref, hh_ref, dd_ref, ds_ref, w_ref, b_ref, o_ref):
    agg = (p_ref[0] + p_ref[1] + hh_ref[...]) * _col0_norm(dd_ref)
    h = jnp.dot(agg, w_ref[...], preferred_element_type=jnp.float32) + b_ref[...]
    h = jnp.maximum(h, 0.0)
    o_ref[...] = h * _col0_norm(ds_ref)


def _final_body(p_ref, hh_ref, dd_ref, wg_ref, bg_ref,
                w1_ref, b1_ref, w2_ref, b2_ref, w3_ref, b3_ref,
                prob_ref, g_ref):
    agg = (p_ref[0] + p_ref[1] + hh_ref[...]) * _col0_norm(dd_ref)
    z = jnp.dot(agg, wg_ref[...], preferred_element_type=jnp.float32) + bg_ref[...]
    g = jax.nn.sigmoid(z) + 1e-8
    g_ref[...] = g
    x = jnp.maximum(jnp.dot(g, w1_ref[...], preferred_element_type=jnp.float32)
                    + b1_ref[...], 0.0)
    x = jnp.maximum(jnp.dot(x, w2_ref[...], preferred_element_type=jnp.float32)
                    + b2_ref[...], 0.0)
    lg = jnp.dot(x, w3_ref[...], preferred_element_type=jnp.float32) + b3_ref[...]
    m = jnp.max(lg, axis=-1, keepdims=True)
    e = jnp.exp(lg - m)
    prob_ref[...] = e / jnp.sum(e, axis=-1, keepdims=True)


def _rows_spec(width):
    return pl.BlockSpec((_BLK, width), lambda i: (i, 0))


def _deg_spec():
    return pl.BlockSpec((2, _BLK, 16), lambda i: (0, i, 0))


def _parts_spec():
    return pl.BlockSpec((2, _BLK, D), lambda i: (0, i, 0))


def _w_spec(r, k):
    return pl.BlockSpec((r, k), lambda i: (0, 0))


def _tc_scale(x, degs):
    return pl.pallas_call(
        _scale_body,
        grid=(_GRID,),
        in_specs=[_rows_spec(D), _deg_spec()],
        out_specs=_rows_spec(D),
        out_shape=jax.ShapeDtypeStruct((N, D), jnp.float32),
    )(x, degs)


def _tc_layer(parts, hh, degd, degs, w, b):
    return pl.pallas_call(
        _layer_body,
        grid=(_GRID,),
        in_specs=[_parts_spec(), _rows_spec(D), _deg_spec(), _deg_spec(),
                  _w_spec(D, D), _w_spec(1, D)],
        out_specs=_rows_spec(D),
        out_shape=jax.ShapeDtypeStruct((N, D), jnp.float32),
    )(parts, hh, degd, degs, w, b)


def _tc_final(parts, hh, degd, wg, bg, w1, b1, w2, b2, w3, b3):
    return pl.pallas_call(
        _final_body,
        grid=(_GRID,),
        in_specs=[_parts_spec(), _rows_spec(D), _deg_spec(),
                  _w_spec(D, D), _w_spec(1, D),
                  _w_spec(D, 256), _w_spec(1, 256),
                  _w_spec(256, 256), _w_spec(1, 256),
                  _w_spec(256, 16), _w_spec(1, 16)],
        out_specs=[_rows_spec(16), _rows_spec(D)],
        out_shape=[jax.ShapeDtypeStruct((N, 16), jnp.float32),
                   jax.ShapeDtypeStruct((N, D), jnp.float32)],
    )(parts, hh, degd, wg, bg, w1, b1, w2, b2, w3, b3)


def kernel(actor_input, edge_index, W_g1, b_g1, W_g2, b_g2, W_g3, b_g3,
           W_f1, b_f1, W_f2, b_f2, W_f3, b_f3):
    src = edge_index[0].astype(jnp.int32)
    dst = edge_index[1].astype(jnp.int32)

    npad = E_PAD - E
    pad = jnp.arange(npad, dtype=jnp.int32)
    src_edge_b = jnp.concatenate([src, pad % N]).reshape(NW, CPW, CHUNK)
    src_deg_b = jnp.concatenate([src, N + pad % NDUMMY]).reshape(NW, CPW, CHUNK)
    dst_b = jnp.concatenate([dst, N + pad % NDUMMY]).reshape(NW, CPW, CHUNK)

    degs, degd = _deg_pass(src_deg_b, dst_b)

    hh1 = _tc_scale(actor_input.astype(jnp.float32), degs)
    p1 = _edge_pass(hh1, src_edge_b, dst_b)
    hh2 = _tc_layer(p1, hh1, degd, degs, W_g1, b_g1.reshape(1, -1))
    p2 = _edge_pass(hh2, src_edge_b, dst_b)
    hh3 = _tc_layer(p2, hh2, degd, degs, W_g2, b_g2.reshape(1, -1))
    p3 = _edge_pass(hh3, src_edge_b, dst_b)
    prob, gnn_output = _tc_final(p3, hh3, degd, W_g3, b_g3.reshape(1, -1),
                                 W_f1, b_f1.reshape(1, -1),
                                 W_f2, b_f2.reshape(1, -1),
                                 W_f3, b_f3.reshape(1, -1))
    return (prob, gnn_output)
